# baseline (speedup 1.0000x reference)
import functools

import jax
import jax.numpy as jnp
from jax import lax
from jax.experimental import pallas as pl
from jax.experimental.pallas import tpu as pltpu
from jax.experimental.pallas import tpu_sc as plsc

N = 16384
D_ATOM = 16
D_CHARGE = 32
D_NUM = 16
D_OUT = D_ATOM + D_CHARGE + D_NUM
CHUNK = 128


def _build(nc, ns):
    nw = nc * ns
    bpw = N // nw
    nchunk = bpw // CHUNK
    mesh = plsc.VectorSubcoreMesh(core_axis_name="c", subcore_axis_name="s")

    @functools.partial(
        pl.kernel,
        mesh=mesh,
        compiler_params=pltpu.CompilerParams(use_tc_tiling_on_sc=False),
        out_type=jax.ShapeDtypeStruct((N, D_OUT), jnp.float32),
        scratch_types=[
            pltpu.VMEM((nchunk, CHUNK), jnp.int32),
            pltpu.VMEM((nchunk, CHUNK), jnp.int32),
            pltpu.VMEM((bpw, D_ATOM), jnp.float32),
            pltpu.VMEM((bpw, D_CHARGE), jnp.float32),
            pltpu.VMEM((bpw, D_NUM), jnp.float32),
            pltpu.VMEM((bpw, D_OUT), jnp.float32),
            pltpu.SemaphoreType.DMA,
        ],
    )
    def k(extra_hbm, wa_hbm, wc_hbm, ia_hbm, ic_hbm, out_hbm,
          ia_v, ic_v, atom_v, charge_v, num_v, out_v, sem):
        wid = lax.axis_index("s") * nc + lax.axis_index("c")
        base = wid * bpw
        pltpu.sync_copy(ia_hbm.at[wid], ia_v)
        pltpu.sync_copy(ic_hbm.at[wid], ic_v)
        copies = []
        for j in range(nchunk):
            rows = pl.ds(j * CHUNK, CHUNK)
            copies.append(
                pltpu.async_copy(wa_hbm.at[ia_v.at[j]], atom_v.at[rows], sem)
            )
            copies.append(
                pltpu.async_copy(wc_hbm.at[ic_v.at[j]], charge_v.at[rows], sem)
            )
        copies.append(
            pltpu.async_copy(extra_hbm.at[pl.ds(base, bpw)], num_v, sem)
        )
        for c in copies:
            c.wait()

        def body(i, carry):
            out_v[i, pl.ds(0, D_ATOM)] = atom_v[i, :]
            out_v[i, pl.ds(D_ATOM, 16)] = charge_v[i, pl.ds(0, 16)]
            out_v[i, pl.ds(D_ATOM + 16, 16)] = charge_v[i, pl.ds(16, 16)]
            out_v[i, pl.ds(D_ATOM + D_CHARGE, D_NUM)] = num_v[i, :]
            return carry

        lax.fori_loop(0, bpw, body, 0)
        pltpu.sync_copy(out_v, out_hbm.at[pl.ds(base, bpw)])

    return k, nw, nchunk


def kernel(pos, extra_feat, W_atom, W_charge, atom_type, charge_state):
    info = plsc.get_sparse_core_info()
    k, nw, nchunk = _build(info.num_cores, info.num_subcores)
    ia = atom_type.reshape(nw, nchunk, CHUNK)
    ic = charge_state.reshape(nw, nchunk, CHUNK)
    out = k(extra_feat, W_atom, W_charge, ia, ic)
    return out.astype(pos.dtype)


# speedup vs baseline: 1.0162x; 1.0162x over previous
import functools

import jax
import jax.numpy as jnp
from jax import lax
from jax.experimental import pallas as pl
from jax.experimental.pallas import tpu as pltpu
from jax.experimental.pallas import tpu_sc as plsc

N = 16384
D_ATOM = 16
D_CHARGE = 32
D_NUM = 16
D_OUT = D_ATOM + D_CHARGE + D_NUM
CHUNK = 128


def _build(nc, ns):
    nw = nc * ns
    bpw = N // nw
    nchunk = bpw // CHUNK
    mesh = plsc.VectorSubcoreMesh(core_axis_name="c", subcore_axis_name="s")

    @functools.partial(
        pl.kernel,
        mesh=mesh,
        compiler_params=pltpu.CompilerParams(use_tc_tiling_on_sc=False),
        out_type=jax.ShapeDtypeStruct((N, D_OUT), jnp.float32),
        scratch_types=[
            pltpu.VMEM((nchunk, CHUNK), jnp.int32),
            pltpu.VMEM((nchunk, CHUNK), jnp.int32),
            pltpu.VMEM((bpw, D_ATOM), jnp.float32),
            pltpu.VMEM((bpw, D_CHARGE), jnp.float32),
            pltpu.VMEM((bpw, D_NUM), jnp.float32),
            pltpu.SemaphoreType.DMA,
        ],
    )
    def k(extra_hbm, wa_hbm, wc_hbm, ia_hbm, ic_hbm, out_hbm,
          ia_v, ic_v, atom_v, charge_v, num_v, sem):
        wid = lax.axis_index("s") * nc + lax.axis_index("c")
        base = wid * bpw
        rows_out = pl.ds(base, bpw)
        pltpu.sync_copy(ia_hbm.at[wid], ia_v)
        pltpu.sync_copy(ic_hbm.at[wid], ic_v)
        copies = []
        for j in range(nchunk):
            rows = pl.ds(j * CHUNK, CHUNK)
            copies.append(
                pltpu.async_copy(wa_hbm.at[ia_v.at[j]], atom_v.at[rows], sem)
            )
            copies.append(
                pltpu.async_copy(wc_hbm.at[ic_v.at[j]], charge_v.at[rows], sem)
            )
        copies.append(
            pltpu.async_copy(extra_hbm.at[rows_out], num_v, sem)
        )
        for c in copies:
            c.wait()
        outs = [
            pltpu.async_copy(
                atom_v, out_hbm.at[rows_out, pl.ds(0, D_ATOM)], sem
            ),
            pltpu.async_copy(
                charge_v, out_hbm.at[rows_out, pl.ds(D_ATOM, D_CHARGE)], sem
            ),
            pltpu.async_copy(
                num_v,
                out_hbm.at[rows_out, pl.ds(D_ATOM + D_CHARGE, D_NUM)],
                sem,
            ),
        ]
        for c in outs:
            c.wait()

    return k, nw, nchunk


def kernel(pos, extra_feat, W_atom, W_charge, atom_type, charge_state):
    info = plsc.get_sparse_core_info()
    k, nw, nchunk = _build(info.num_cores, info.num_subcores)
    ia = atom_type.reshape(nw, nchunk, CHUNK)
    ic = charge_state.reshape(nw, nchunk, CHUNK)
    out = k(extra_feat, W_atom, W_charge, ia, ic)
    return out.astype(pos.dtype)
